# Initial kernel scaffold; baseline (speedup 1.0000x reference)
import jax
import jax.numpy as jnp
from jax.experimental import pallas as pl


def kernel(x, edge_index, edge_attr, W1m, b1m, W2m, b2m, W1u, b1u, W2u, b2u):
    raise NotImplementedError("write your pallas kernel here")


# speedup vs baseline: 1.6575x; 1.6575x over previous
import functools

import jax
import jax.numpy as jnp
from jax import lax
from jax.experimental import pallas as pl
from jax.experimental.pallas import tpu as pltpu
from jax.experimental.pallas import tpu_sc as plsc

N = 10000
E = 160000
D = 256
DE = 16

NC = 2
NS = 16
NW = NC * NS
CH = 128
EP = 163840
EPW = EP // NW
DH = D // 2
NP = 10240
ZROWS = NP // NS

_f32 = jnp.float32


def _proj_body(x_ref, wa_ref, wb_ref, pa_ref, pb_ref):
    xb = x_ref[...]
    pa_ref[...] = jnp.dot(xb, wa_ref[...], preferred_element_type=_f32)
    pb_ref[...] = jnp.dot(xb, wb_ref[...], preferred_element_type=_f32)


def _node_proj(x, wa, wb):
    BN = 2000
    return pl.pallas_call(
        _proj_body,
        grid=(N // BN,),
        in_specs=[
            pl.BlockSpec((BN, D), lambda i: (i, 0)),
            pl.BlockSpec((D, D), lambda i: (0, 0)),
            pl.BlockSpec((D, D), lambda i: (0, 0)),
        ],
        out_specs=[
            pl.BlockSpec((BN, D), lambda i: (i, 0)),
            pl.BlockSpec((BN, D), lambda i: (i, 0)),
        ],
        out_shape=[jax.ShapeDtypeStruct((N, D), _f32)] * 2,
    )(x, wa, wb)


def _msg_body(ga_ref, gb_ref, ea_ref, w1e_ref, b1_ref, w2_ref, b2_ref, m_ref):
    i = pl.program_id(0)
    be = ga_ref.shape[0]
    pre = (
        ga_ref[...]
        + gb_ref[...]
        + jnp.dot(ea_ref[...], w1e_ref[...], preferred_element_type=_f32)
        + b1_ref[...]
    )
    h = jnp.maximum(pre, 0.0)
    m = jnp.dot(h, w2_ref[...], preferred_element_type=_f32) + b2_ref[...]
    rowid = i * be + lax.broadcasted_iota(jnp.int32, (be, 1), 0)
    m_ref[...] = jnp.where(rowid < E, m, 0.0)


def _messages(ga, gb, ea, w1e, b1, w2, b2):
    BE = 2048
    return pl.pallas_call(
        _msg_body,
        grid=(EP // BE,),
        in_specs=[
            pl.BlockSpec((BE, D), lambda i: (i, 0)),
            pl.BlockSpec((BE, D), lambda i: (i, 0)),
            pl.BlockSpec((BE, DE), lambda i: (i, 0)),
            pl.BlockSpec((DE, D), lambda i: (0, 0)),
            pl.BlockSpec((1, D), lambda i: (0, 0)),
            pl.BlockSpec((D, D), lambda i: (0, 0)),
            pl.BlockSpec((1, D), lambda i: (0, 0)),
        ],
        out_specs=pl.BlockSpec((BE, D), lambda i: (i, 0)),
        out_shape=jax.ShapeDtypeStruct((EP, D), _f32),
    )(ga, gb, ea, w1e, b1, w2, b2)


def _upd_body(x_ref, ag_ref, wua_ref, wub_ref, b1_ref, w2_ref, b2_ref, o_ref):
    pre = (
        jnp.dot(x_ref[...], wua_ref[...], preferred_element_type=_f32)
        + jnp.dot(ag_ref[...], wub_ref[...], preferred_element_type=_f32)
        + b1_ref[...]
    )
    u = jnp.maximum(pre, 0.0)
    o_ref[...] = jnp.dot(u, w2_ref[...], preferred_element_type=_f32) + b2_ref[...]


def _update(x, aggr, wua, wub, b1, w2, b2):
    BN = 2000
    return pl.pallas_call(
        _upd_body,
        grid=(N // BN,),
        in_specs=[
            pl.BlockSpec((BN, D), lambda i: (i, 0)),
            pl.BlockSpec((BN, D), lambda i: (i, 0)),
            pl.BlockSpec((D, D), lambda i: (0, 0)),
            pl.BlockSpec((D, D), lambda i: (0, 0)),
            pl.BlockSpec((1, D), lambda i: (0, 0)),
            pl.BlockSpec((D, D), lambda i: (0, 0)),
            pl.BlockSpec((1, D), lambda i: (0, 0)),
        ],
        out_specs=pl.BlockSpec((BN, D), lambda i: (i, 0)),
        out_shape=jax.ShapeDtypeStruct((N, D), _f32),
    )(x, aggr, wua, wub, b1, w2, b2)


def _gather_body(pa_hbm, pb_hbm, row_hbm, col_hbm, ga_hbm, gb_hbm,
                 idxa_v, idxb_v, rowsa_v, rowsb_v, sem_a, sem_b):
    c = lax.axis_index("c")
    s = lax.axis_index("s")
    wid = s * NC + c
    base = wid * EPW

    def step(i, carry):
        off = base + i * CH
        pltpu.sync_copy(row_hbm.at[pl.ds(off, CH)], idxa_v)
        pltpu.sync_copy(col_hbm.at[pl.ds(off, CH)], idxb_v)
        cpa = pltpu.async_copy(pa_hbm.at[idxa_v], rowsa_v, sem_a)
        cpb = pltpu.async_copy(pb_hbm.at[idxb_v], rowsb_v, sem_b)
        cpa.wait()
        pltpu.sync_copy(rowsa_v, ga_hbm.at[pl.ds(off, CH)])
        cpb.wait()
        pltpu.sync_copy(rowsb_v, gb_hbm.at[pl.ds(off, CH)])
        return carry

    lax.fori_loop(0, EPW // CH, step, 0)


_sc_gather = functools.partial(
    pl.kernel,
    out_type=[jax.ShapeDtypeStruct((EP, D), _f32)] * 2,
    mesh=plsc.VectorSubcoreMesh(core_axis_name="c", subcore_axis_name="s"),
    scratch_types=[
        pltpu.VMEM((CH,), jnp.int32),
        pltpu.VMEM((CH,), jnp.int32),
        pltpu.VMEM((CH, D), _f32),
        pltpu.VMEM((CH, D), _f32),
        pltpu.SemaphoreType.DMA,
        pltpu.SemaphoreType.DMA,
    ],
)(_gather_body)


def _scatter_body(m_hbm, row_hbm, zero_hbm, aggr_hbm,
                  idx_v, m_v, shared):
    c = lax.axis_index("c")
    s = lax.axis_index("s")

    pltpu.sync_copy(zero_hbm, m_v)
    for k in range(ZROWS // CH):
        pltpu.sync_copy(m_v, shared.at[pl.ds(s * ZROWS + k * CH, CH)])
    plsc.subcore_barrier()

    nchunks = EP // CH // NS

    def step(i, carry):
        off = (s * nchunks + i) * CH
        pltpu.sync_copy(row_hbm.at[pl.ds(off, CH)], idx_v)
        pltpu.sync_copy(m_hbm.at[pl.ds(off, CH), pl.ds(c * DH, DH)], m_v)
        pltpu.sync_copy(m_v, shared.at[idx_v], add=True)
        return carry

    lax.fori_loop(0, nchunks, step, 0)
    plsc.subcore_barrier()

    pltpu.sync_copy(
        shared.at[pl.ds(s * ZROWS, ZROWS)],
        aggr_hbm.at[pl.ds(s * ZROWS, ZROWS), pl.ds(c * DH, DH)],
    )


_sc_scatter = functools.partial(
    pl.kernel,
    out_type=jax.ShapeDtypeStruct((NP, D), _f32),
    mesh=plsc.VectorSubcoreMesh(core_axis_name="c", subcore_axis_name="s"),
    scratch_types=[
        pltpu.VMEM((CH,), jnp.int32),
        pltpu.VMEM((CH, DH), _f32),
        pltpu.VMEM_SHARED((NP, DH), _f32),
    ],
)(_scatter_body)


def kernel(x, edge_index, edge_attr, W1m, b1m, W2m, b2m, W1u, b1u, W2u, b2u):
    row = edge_index[0].astype(jnp.int32)
    col = edge_index[1].astype(jnp.int32)
    rowp = jnp.pad(row, (0, EP - E))
    colp = jnp.pad(col, (0, EP - E))
    eap = jnp.pad(edge_attr, ((0, EP - E), (0, 0)))

    W1a, W1b, W1e = W1m[:D], W1m[D:2 * D], W1m[2 * D:]
    Wua, Wub = W1u[:D], W1u[D:]

    pa, pb = _node_proj(x, W1a, W1b)
    ga, gb = _sc_gather(pa, pb, rowp, colp)
    msgs = _messages(ga, gb, eap, W1e, b1m.reshape(1, D), W2m,
                     b2m.reshape(1, D))
    zero = jnp.zeros((CH, DH), _f32)
    aggr = _sc_scatter(msgs, rowp, zero)
    out = _update(x, aggr, Wua, Wub, b1u.reshape(1, D), W2u, b2u.reshape(1, D))
    return out
